# Initial kernel scaffold; baseline (speedup 1.0000x reference)
import jax
import jax.numpy as jnp
from jax.experimental import pallas as pl


def kernel(var_learned_f, var_lp_f, con_learned_f, con_lp_f, edge_learned_f, solver_state, edge_lp_f_wo_ss, edge_index_var_con, params):
    raise NotImplementedError("write your pallas kernel here")


# speedup vs baseline: 2.2247x; 2.2247x over previous
import functools
import math

import jax
import jax.numpy as jnp
from jax.experimental import pallas as pl


def _linear(x, W, b):
    return x @ W + b


def _relu_pallas(x):
    n, c = x.shape
    tile = 2000
    assert n % tile == 0

    def body(x_ref, o_ref):
        o_ref[...] = jnp.maximum(x_ref[...], 0.0)

    return pl.pallas_call(
        body,
        grid=(n // tile,),
        in_specs=[pl.BlockSpec((tile, c), lambda i: (i, 0))],
        out_specs=pl.BlockSpec((tile, c), lambda i: (i, 0)),
        out_shape=jax.ShapeDtypeStruct((n, c), x.dtype),
    )(x)


def _conv(p, x_src, x_dst, e_src, e_dst, edge_attr, cnt_dst):
    C = p['Wq'].shape[1]
    q = _linear(x_dst, p['Wq'], p['bq'])
    k = _linear(x_src, p['Wk'], p['bk'])
    v = _linear(x_src, p['Wv'], p['bv'])
    e = _linear(edge_attr, p['We'], p['be'])
    kj = k[e_src] + e
    vj = v[e_src] + e
    alpha = jnp.sum(q[e_dst] * kj, axis=-1) / math.sqrt(C)
    w = jnp.exp(alpha)
    n_dst = x_dst.shape[0]
    S = jax.ops.segment_sum(w[:, None] * vj, e_dst, num_segments=n_dst)
    W = jax.ops.segment_sum(w, e_dst, num_segments=n_dst)
    out = S / jnp.maximum(W, 1e-16)[:, None] / jnp.maximum(cnt_dst, 1.0)[:, None]
    out = out + _linear(x_dst, p['Ws'], p['bs'])
    return _relu_pallas(out)


def _mlp2(x, p):
    h = jax.nn.relu(_linear(x, p['W1'], p['b1']))
    return jax.nn.relu(_linear(h, p['W2'], p['b2']))


def kernel(var_learned_f, var_lp_f, con_learned_f, con_lp_f, edge_learned_f,
           solver_state, edge_lp_f_wo_ss, edge_index_var_con, params):
    ei = edge_index_var_con
    vi, ci = ei[0], ei[1]
    Nv = var_learned_f.shape[0]
    Nc = con_learned_f.shape[0]
    E = vi.shape[0]

    var_comb = jnp.concatenate([var_learned_f, var_lp_f], axis=1)
    con_comb = jnp.concatenate([con_learned_f, con_lp_f], axis=1)
    edge_comb = jnp.concatenate([edge_learned_f, edge_lp_f_wo_ss], axis=1)

    ones = jnp.ones((E,), jnp.float32)
    cnt_con = jax.ops.segment_sum(ones, ci, num_segments=Nc)
    cnt_var = jax.ops.segment_sum(ones, vi, num_segments=Nv)

    con_new = _conv(params['con_upd'], var_comb, con_comb, vi, ci, edge_comb, cnt_con)
    con_comb2 = jnp.concatenate([con_new, con_lp_f], axis=1)
    var_new = _conv(params['var_upd'], con_comb2, var_comb, ci, vi, edge_comb, cnt_var)
    var_comb2 = jnp.concatenate([var_new, var_lp_f], axis=1)

    ep = params['edge_upd']
    vc = _mlp2(var_comb2, ep['var_mlp'])
    cc = _mlp2(con_comb2, ep['con_mlp'])
    W1 = ep['edge_mlp']['W1']
    W1e, W1v, W1c = W1[:12], W1[12:20], W1[20:28]
    pv = vc @ W1v
    pc = cc @ W1c
    h = jax.nn.relu(edge_comb @ W1e + pv[vi] + pc[ci] + ep['edge_mlp']['b1'])
    edge_new = jax.nn.relu(_linear(h, ep['edge_mlp']['W2'], ep['edge_mlp']['b2']))
    return (var_new, con_new, edge_new)


# speedup vs baseline: 2.7827x; 1.2508x over previous
import functools
import math

import jax
import jax.numpy as jnp
from jax import lax
from jax.experimental import pallas as pl
from jax.experimental.pallas import tpu as pltpu
from jax.experimental.pallas import tpu_sc as plsc

NP_NODES = 100352
NWORKERS = 32
F32 = jnp.float32


def _mesh():
    return plsc.VectorSubcoreMesh(core_axis_name="c", subcore_axis_name="s")


def _wid():
    return lax.axis_index("s") * 2 + lax.axis_index("c")


def _sc_scatter(payload, w, idx, E, B=1000):
    epw = E // NWORKERS
    assert epw % B == 0
    n_chunks = epw // B
    rows_per_tile = NP_NODES // 16

    @functools.partial(
        pl.kernel,
        out_type=[
            jax.ShapeDtypeStruct((2 * NP_NODES, 16), F32),
            jax.ShapeDtypeStruct((2 * NP_NODES,), F32),
        ],
        scratch_types=[
            pltpu.VMEM_SHARED((NP_NODES, 16), F32),
            pltpu.VMEM_SHARED((NP_NODES,), F32),
            pltpu.VMEM((B, 16), F32),
            pltpu.VMEM((B,), F32),
            pltpu.VMEM((B,), jnp.int32),
        ],
        mesh=_mesh(),
        compiler_params=pltpu.CompilerParams(use_tc_tiling_on_sc=False),
    )
    def body(payload_hbm, w_hbm, idx_hbm, s_out, w_out,
             acc, accw, pbuf, wbuf, ibuf):
        cid = lax.axis_index("c")
        sid = lax.axis_index("s")
        wid = sid * 2 + cid

        def zrow(i, _):
            pbuf[i] = jnp.zeros((16,), F32)
            return 0
        lax.fori_loop(0, B, zrow, 0)

        def zrow2(i, _):
            wbuf[pl.ds(i * 16, 16)] = jnp.zeros((16,), F32)
            return 0
        lax.fori_loop(0, B // 8 // 2, zrow2, 0)
        wbuf[pl.ds(B - 16, 16)] = jnp.zeros((16,), F32)

        r0 = sid * rows_per_tile
        for z0 in (0, B, 2 * B, 3 * B, 4 * B, 5 * B, rows_per_tile - B):
            pltpu.sync_copy(pbuf, acc.at[pl.ds(r0 + z0, B)])
            pltpu.sync_copy(wbuf, accw.at[pl.ds(r0 + z0, B)])
        plsc.subcore_barrier()

        def chunk(j, _):
            off = wid * epw + j * B
            pltpu.sync_copy(payload_hbm.at[pl.ds(off, B)], pbuf)
            pltpu.sync_copy(w_hbm.at[pl.ds(off, B)], wbuf)
            pltpu.sync_copy(idx_hbm.at[pl.ds(off, B)], ibuf)
            pltpu.sync_copy(pbuf, acc.at[ibuf], add=True)
            pltpu.sync_copy(wbuf, accw.at[ibuf], add=True)
            return 0
        lax.fori_loop(0, n_chunks, chunk, 0)

        plsc.subcore_barrier()
        o0 = cid * NP_NODES + r0
        pltpu.sync_copy(acc.at[pl.ds(r0, rows_per_tile)],
                        s_out.at[pl.ds(o0, rows_per_tile)])
        pltpu.sync_copy(accw.at[pl.ds(r0, rows_per_tile)],
                        w_out.at[pl.ds(o0, rows_per_tile)])

    s2, w2 = body(payload, w, idx)
    return s2.reshape(2, NP_NODES, 16), w2.reshape(2, NP_NODES)


def _sc_counts(vi, ci, E, B=2000):
    epw = E // NWORKERS
    assert epw % B == 0
    n_chunks = epw // B
    rows_per_tile = NP_NODES // 16

    @functools.partial(
        pl.kernel,
        out_type=[
            jax.ShapeDtypeStruct((2 * NP_NODES,), F32),
            jax.ShapeDtypeStruct((2 * NP_NODES,), F32),
        ],
        scratch_types=[
            pltpu.VMEM_SHARED((NP_NODES,), F32),
            pltpu.VMEM_SHARED((NP_NODES,), F32),
            pltpu.VMEM((B,), F32),
            pltpu.VMEM((B,), F32),
            pltpu.VMEM((B,), jnp.int32),
            pltpu.VMEM((B,), jnp.int32),
        ],
        mesh=_mesh(),
        compiler_params=pltpu.CompilerParams(use_tc_tiling_on_sc=False),
    )
    def body(vi_hbm, ci_hbm, cv_out, cc_out,
             accv, accc, zeros_b, ones, ivbuf, icbuf):
        cid = lax.axis_index("c")
        sid = lax.axis_index("s")
        wid = sid * 2 + cid

        def zrow(i, _):
            zeros_b[pl.ds(i * 16, 16)] = jnp.zeros((16,), F32)
            ones[pl.ds(i * 16, 16)] = jnp.ones((16,), F32)
            return 0
        lax.fori_loop(0, B // 16, zrow, 0)

        r0 = sid * rows_per_tile
        for z0 in (0, B, 2 * B, rows_per_tile - B):
            pltpu.sync_copy(zeros_b, accv.at[pl.ds(r0 + z0, B)])
            pltpu.sync_copy(zeros_b, accc.at[pl.ds(r0 + z0, B)])
        plsc.subcore_barrier()

        def chunk(j, _):
            off = wid * epw + j * B
            pltpu.sync_copy(vi_hbm.at[pl.ds(off, B)], ivbuf)
            pltpu.sync_copy(ci_hbm.at[pl.ds(off, B)], icbuf)
            pltpu.sync_copy(ones, accv.at[ivbuf], add=True)
            pltpu.sync_copy(ones, accc.at[icbuf], add=True)
            return 0
        lax.fori_loop(0, n_chunks, chunk, 0)

        plsc.subcore_barrier()
        o0 = cid * NP_NODES + r0
        pltpu.sync_copy(accv.at[pl.ds(r0, rows_per_tile)],
                        cv_out.at[pl.ds(o0, rows_per_tile)])
        pltpu.sync_copy(accc.at[pl.ds(r0, rows_per_tile)],
                        cc_out.at[pl.ds(o0, rows_per_tile)])

    cv2, cc2 = body(vi, ci)
    return cv2.reshape(2, NP_NODES), cc2.reshape(2, NP_NODES)


_DEBUG_XLA_SCATTER = False


def _linear(x, W, b):
    return x @ W + b


def _conv(p, x_src, x_dst, e_src, e_dst, edge_attr, winv_den):
    C = p['Wq'].shape[1]
    q = _linear(x_dst, p['Wq'], p['bq'])
    k = _linear(x_src, p['Wk'], p['bk'])
    v = _linear(x_src, p['Wv'], p['bv'])
    e = _linear(edge_attr, p['We'], p['be'])
    kj = k[e_src] + e
    vj = v[e_src] + e
    alpha = jnp.sum(q[e_dst] * kj, axis=-1) / math.sqrt(C)
    w = jnp.exp(alpha)
    payload = w[:, None] * vj

    n_dst = x_dst.shape[0]
    if _DEBUG_XLA_SCATTER:
        S = jax.ops.segment_sum(payload, e_dst, num_segments=n_dst)
        Wd = jax.ops.segment_sum(w, e_dst, num_segments=n_dst)
    else:
        Sp, Wp = _sc_scatter(payload, w, e_dst, e_src.shape[0])
        S = (Sp[0] + Sp[1])[:n_dst]
        Wd = (Wp[0] + Wp[1])[:n_dst]
    out = S / (jnp.maximum(Wd, 1e-16) * winv_den[:n_dst])[:, None]
    out = out + _linear(x_dst, p['Ws'], p['bs'])
    return jax.nn.relu(out)


def _mlp2(x, p):
    h = jax.nn.relu(_linear(x, p['W1'], p['b1']))
    return jax.nn.relu(_linear(h, p['W2'], p['b2']))


def kernel(var_learned_f, var_lp_f, con_learned_f, con_lp_f, edge_learned_f,
           solver_state, edge_lp_f_wo_ss, edge_index_var_con, params):
    ei = edge_index_var_con
    vi, ci = ei[0], ei[1]

    var_comb = jnp.concatenate([var_learned_f, var_lp_f], axis=1)
    con_comb = jnp.concatenate([con_learned_f, con_lp_f], axis=1)
    edge_comb = jnp.concatenate([edge_learned_f, edge_lp_f_wo_ss], axis=1)

    Cvp, Ccp = _sc_counts(vi, ci, vi.shape[0])
    cnt_var = jnp.maximum(Cvp[0] + Cvp[1], 1.0)
    cnt_con = jnp.maximum(Ccp[0] + Ccp[1], 1.0)

    con_new = _conv(params['con_upd'], var_comb, con_comb, vi, ci, edge_comb, cnt_con)
    con_comb2 = jnp.concatenate([con_new, con_lp_f], axis=1)
    var_new = _conv(params['var_upd'], con_comb2, var_comb, ci, vi, edge_comb, cnt_var)
    var_comb2 = jnp.concatenate([var_new, var_lp_f], axis=1)

    ep = params['edge_upd']
    vc = _mlp2(var_comb2, ep['var_mlp'])
    cc = _mlp2(con_comb2, ep['con_mlp'])
    W1 = ep['edge_mlp']['W1']
    W1e, W1v, W1c = W1[:12], W1[12:20], W1[20:28]
    pv = vc @ W1v
    pc = cc @ W1c
    h = jax.nn.relu(edge_comb @ W1e + pv[vi] + pc[ci] + ep['edge_mlp']['b1'])
    edge_new = jax.nn.relu(_linear(h, ep['edge_mlp']['W2'], ep['edge_mlp']['b2']))
    return (var_new, con_new, edge_new)


# speedup vs baseline: 9.0948x; 3.2684x over previous
import functools
import math

import jax
import jax.numpy as jnp
from jax import lax
from jax.experimental import pallas as pl
from jax.experimental.pallas import tpu as pltpu
from jax.experimental.pallas import tpu_sc as plsc

NP_NODES = 100352
NWORKERS = 32
F32 = jnp.float32


def _mesh():
    return plsc.VectorSubcoreMesh(core_axis_name="c", subcore_axis_name="s")


def _wid():
    return lax.axis_index("s") * 2 + lax.axis_index("c")


def _sc_scatter(payload, w, idx, E, B=1000):
    epw = E // NWORKERS
    assert epw % B == 0
    n_chunks = epw // B
    rows_per_tile = NP_NODES // 16

    @functools.partial(
        pl.kernel,
        out_type=[
            jax.ShapeDtypeStruct((2 * NP_NODES, 16), F32),
            jax.ShapeDtypeStruct((2 * NP_NODES,), F32),
        ],
        scratch_types=[
            pltpu.VMEM_SHARED((NP_NODES, 16), F32),
            pltpu.VMEM_SHARED((NP_NODES,), F32),
            pltpu.VMEM((B, 16), F32),
            pltpu.VMEM((B,), F32),
            pltpu.VMEM((B,), jnp.int32),
        ],
        mesh=_mesh(),
        compiler_params=pltpu.CompilerParams(use_tc_tiling_on_sc=False),
    )
    def body(payload_hbm, w_hbm, idx_hbm, s_out, w_out,
             acc, accw, pbuf, wbuf, ibuf):
        cid = lax.axis_index("c")
        sid = lax.axis_index("s")
        wid = sid * 2 + cid

        def zrow(i, _):
            pbuf[i] = jnp.zeros((16,), F32)
            return 0
        lax.fori_loop(0, B, zrow, 0)

        def zrow2(i, _):
            wbuf[pl.ds(i * 16, 16)] = jnp.zeros((16,), F32)
            return 0
        lax.fori_loop(0, B // 8 // 2, zrow2, 0)
        wbuf[pl.ds(B - 16, 16)] = jnp.zeros((16,), F32)

        r0 = sid * rows_per_tile
        for z0 in (0, B, 2 * B, 3 * B, 4 * B, 5 * B, rows_per_tile - B):
            pltpu.sync_copy(pbuf, acc.at[pl.ds(r0 + z0, B)])
            pltpu.sync_copy(wbuf, accw.at[pl.ds(r0 + z0, B)])
        plsc.subcore_barrier()

        def chunk(j, _):
            off = wid * epw + j * B
            pltpu.sync_copy(payload_hbm.at[pl.ds(off, B)], pbuf)
            pltpu.sync_copy(w_hbm.at[pl.ds(off, B)], wbuf)
            pltpu.sync_copy(idx_hbm.at[pl.ds(off, B)], ibuf)
            pltpu.sync_copy(pbuf, acc.at[ibuf], add=True)
            pltpu.sync_copy(wbuf, accw.at[ibuf], add=True)
            return 0
        lax.fori_loop(0, n_chunks, chunk, 0)

        plsc.subcore_barrier()
        o0 = cid * NP_NODES + r0
        pltpu.sync_copy(acc.at[pl.ds(r0, rows_per_tile)],
                        s_out.at[pl.ds(o0, rows_per_tile)])
        pltpu.sync_copy(accw.at[pl.ds(r0, rows_per_tile)],
                        w_out.at[pl.ds(o0, rows_per_tile)])

    s2, w2 = body(payload, w, idx)
    return s2.reshape(2, NP_NODES, 16), w2.reshape(2, NP_NODES)


def _sc_gather(table, idx, B=1000):
    E = idx.shape[0]
    N, D = table.shape
    epw = E // NWORKERS
    assert epw % B == 0
    n_chunks = epw // B

    @functools.partial(
        pl.kernel,
        out_type=jax.ShapeDtypeStruct((E, D), F32),
        scratch_types=[
            pltpu.VMEM((B,), jnp.int32),
            pltpu.VMEM((B, D), F32),
            pltpu.SemaphoreType.DMA,
        ],
        mesh=_mesh(),
        compiler_params=pltpu.CompilerParams(use_tc_tiling_on_sc=False),
    )
    def body(table_hbm, idx_hbm, out_hbm, ibuf, rbuf, sem):
        wid = _wid()

        def chunk(j, _):
            off = wid * epw + j * B
            pltpu.sync_copy(idx_hbm.at[pl.ds(off, B)], ibuf)
            pltpu.async_copy(table_hbm.at[ibuf], rbuf, sem).wait()
            pltpu.sync_copy(rbuf, out_hbm.at[pl.ds(off, B)])
            return 0
        lax.fori_loop(0, n_chunks, chunk, 0)

    return body(table, idx)


def _sc_counts(vi, ci, E, B=2000):
    epw = E // NWORKERS
    assert epw % B == 0
    n_chunks = epw // B
    rows_per_tile = NP_NODES // 16

    @functools.partial(
        pl.kernel,
        out_type=[
            jax.ShapeDtypeStruct((2 * NP_NODES,), F32),
            jax.ShapeDtypeStruct((2 * NP_NODES,), F32),
        ],
        scratch_types=[
            pltpu.VMEM_SHARED((NP_NODES,), F32),
            pltpu.VMEM_SHARED((NP_NODES,), F32),
            pltpu.VMEM((B,), F32),
            pltpu.VMEM((B,), F32),
            pltpu.VMEM((B,), jnp.int32),
            pltpu.VMEM((B,), jnp.int32),
        ],
        mesh=_mesh(),
        compiler_params=pltpu.CompilerParams(use_tc_tiling_on_sc=False),
    )
    def body(vi_hbm, ci_hbm, cv_out, cc_out,
             accv, accc, zeros_b, ones, ivbuf, icbuf):
        cid = lax.axis_index("c")
        sid = lax.axis_index("s")
        wid = sid * 2 + cid

        def zrow(i, _):
            zeros_b[pl.ds(i * 16, 16)] = jnp.zeros((16,), F32)
            ones[pl.ds(i * 16, 16)] = jnp.ones((16,), F32)
            return 0
        lax.fori_loop(0, B // 16, zrow, 0)

        r0 = sid * rows_per_tile
        for z0 in (0, B, 2 * B, rows_per_tile - B):
            pltpu.sync_copy(zeros_b, accv.at[pl.ds(r0 + z0, B)])
            pltpu.sync_copy(zeros_b, accc.at[pl.ds(r0 + z0, B)])
        plsc.subcore_barrier()

        def chunk(j, _):
            off = wid * epw + j * B
            pltpu.sync_copy(vi_hbm.at[pl.ds(off, B)], ivbuf)
            pltpu.sync_copy(ci_hbm.at[pl.ds(off, B)], icbuf)
            pltpu.sync_copy(ones, accv.at[ivbuf], add=True)
            pltpu.sync_copy(ones, accc.at[icbuf], add=True)
            return 0
        lax.fori_loop(0, n_chunks, chunk, 0)

        plsc.subcore_barrier()
        o0 = cid * NP_NODES + r0
        pltpu.sync_copy(accv.at[pl.ds(r0, rows_per_tile)],
                        cv_out.at[pl.ds(o0, rows_per_tile)])
        pltpu.sync_copy(accc.at[pl.ds(r0, rows_per_tile)],
                        cc_out.at[pl.ds(o0, rows_per_tile)])

    cv2, cc2 = body(vi, ci)
    return cv2.reshape(2, NP_NODES), cc2.reshape(2, NP_NODES)


_DEBUG_XLA_SCATTER = False


def _linear(x, W, b):
    return x @ W + b


def _conv(p, x_src, x_dst, e_src, e_dst, edge_attr, winv_den):
    C = p['Wq'].shape[1]
    q = _linear(x_dst, p['Wq'], p['bq'])
    kv = jnp.concatenate([
        _linear(x_src, p['Wk'], p['bk']),
        _linear(x_src, p['Wv'], p['bv'])], axis=1)
    e = _linear(edge_attr, p['We'], p['be'])
    kvj = _sc_gather(kv, e_src)
    qd = _sc_gather(q, e_dst)
    kj = kvj[:, :16] + e
    vj = kvj[:, 16:] + e
    alpha = jnp.sum(qd * kj, axis=-1) / math.sqrt(C)
    w = jnp.exp(alpha)
    payload = w[:, None] * vj

    n_dst = x_dst.shape[0]
    if _DEBUG_XLA_SCATTER:
        S = jax.ops.segment_sum(payload, e_dst, num_segments=n_dst)
        Wd = jax.ops.segment_sum(w, e_dst, num_segments=n_dst)
    else:
        Sp, Wp = _sc_scatter(payload, w, e_dst, e_src.shape[0])
        S = (Sp[0] + Sp[1])[:n_dst]
        Wd = (Wp[0] + Wp[1])[:n_dst]
    out = S / (jnp.maximum(Wd, 1e-16) * winv_den[:n_dst])[:, None]
    out = out + _linear(x_dst, p['Ws'], p['bs'])
    return jax.nn.relu(out)


def _mlp2(x, p):
    h = jax.nn.relu(_linear(x, p['W1'], p['b1']))
    return jax.nn.relu(_linear(h, p['W2'], p['b2']))


def kernel(var_learned_f, var_lp_f, con_learned_f, con_lp_f, edge_learned_f,
           solver_state, edge_lp_f_wo_ss, edge_index_var_con, params):
    ei = edge_index_var_con
    vi, ci = ei[0], ei[1]

    var_comb = jnp.concatenate([var_learned_f, var_lp_f], axis=1)
    con_comb = jnp.concatenate([con_learned_f, con_lp_f], axis=1)
    edge_comb = jnp.concatenate([edge_learned_f, edge_lp_f_wo_ss], axis=1)

    Cvp, Ccp = _sc_counts(vi, ci, vi.shape[0])
    cnt_var = jnp.maximum(Cvp[0] + Cvp[1], 1.0)
    cnt_con = jnp.maximum(Ccp[0] + Ccp[1], 1.0)

    con_new = _conv(params['con_upd'], var_comb, con_comb, vi, ci, edge_comb, cnt_con)
    con_comb2 = jnp.concatenate([con_new, con_lp_f], axis=1)
    var_new = _conv(params['var_upd'], con_comb2, var_comb, ci, vi, edge_comb, cnt_var)
    var_comb2 = jnp.concatenate([var_new, var_lp_f], axis=1)

    ep = params['edge_upd']
    vc = _mlp2(var_comb2, ep['var_mlp'])
    cc = _mlp2(con_comb2, ep['con_mlp'])
    W1 = ep['edge_mlp']['W1']
    W1e, W1v, W1c = W1[:12], W1[12:20], W1[20:28]
    pv = jnp.pad(vc @ W1v, ((0, 0), (0, 8)))
    pc = jnp.pad(cc @ W1c, ((0, 0), (0, 8)))
    pvj = _sc_gather(pv, vi)[:, :8]
    pcj = _sc_gather(pc, ci)[:, :8]
    h = jax.nn.relu(edge_comb @ W1e + pvj + pcj + ep['edge_mlp']['b1'])
    edge_new = jax.nn.relu(_linear(h, ep['edge_mlp']['W2'], ep['edge_mlp']['b2']))
    return (var_new, con_new, edge_new)


# speedup vs baseline: 14.7916x; 1.6264x over previous
import functools
import math

import jax
import jax.numpy as jnp
from jax import lax
from jax.experimental import pallas as pl
from jax.experimental.pallas import tpu as pltpu
from jax.experimental.pallas import tpu_sc as plsc

NP_NODES = 100352
NWORKERS = 32
F32 = jnp.float32


def _mesh():
    return plsc.VectorSubcoreMesh(core_axis_name="c", subcore_axis_name="s")


def _wid():
    return lax.axis_index("s") * 2 + lax.axis_index("c")


def _sc_scatter(payload, w, idx, E, B=1000):
    epw = E // NWORKERS
    assert epw % B == 0
    n_chunks = epw // B
    rows_per_tile = NP_NODES // 16

    @functools.partial(
        pl.kernel,
        out_type=[
            jax.ShapeDtypeStruct((2 * NP_NODES, 16), F32),
            jax.ShapeDtypeStruct((2 * NP_NODES,), F32),
        ],
        scratch_types=[
            pltpu.VMEM_SHARED((NP_NODES, 16), F32),
            pltpu.VMEM_SHARED((NP_NODES,), F32),
            pltpu.VMEM((B, 16), F32),
            pltpu.VMEM((B,), F32),
            pltpu.VMEM((B,), jnp.int32),
        ],
        mesh=_mesh(),
        compiler_params=pltpu.CompilerParams(use_tc_tiling_on_sc=False),
    )
    def body(payload_hbm, w_hbm, idx_hbm, s_out, w_out,
             acc, accw, pbuf, wbuf, ibuf):
        cid = lax.axis_index("c")
        sid = lax.axis_index("s")
        wid = sid * 2 + cid

        def zrow(i, _):
            pbuf[i] = jnp.zeros((16,), F32)
            return 0
        lax.fori_loop(0, B, zrow, 0)

        def zrow2(i, _):
            wbuf[pl.ds(i * 16, 16)] = jnp.zeros((16,), F32)
            return 0
        lax.fori_loop(0, B // 8 // 2, zrow2, 0)
        wbuf[pl.ds(B - 16, 16)] = jnp.zeros((16,), F32)

        r0 = sid * rows_per_tile
        for z0 in (0, B, 2 * B, 3 * B, 4 * B, 5 * B, rows_per_tile - B):
            pltpu.sync_copy(pbuf, acc.at[pl.ds(r0 + z0, B)])
            pltpu.sync_copy(wbuf, accw.at[pl.ds(r0 + z0, B)])
        plsc.subcore_barrier()

        def chunk(j, _):
            off = wid * epw + j * B
            pltpu.sync_copy(payload_hbm.at[pl.ds(off, B)], pbuf)
            pltpu.sync_copy(w_hbm.at[pl.ds(off, B)], wbuf)
            pltpu.sync_copy(idx_hbm.at[pl.ds(off, B)], ibuf)
            pltpu.sync_copy(pbuf, acc.at[ibuf], add=True)
            pltpu.sync_copy(wbuf, accw.at[ibuf], add=True)
            return 0
        lax.fori_loop(0, n_chunks, chunk, 0)

        plsc.subcore_barrier()
        o0 = cid * NP_NODES + r0
        pltpu.sync_copy(acc.at[pl.ds(r0, rows_per_tile)],
                        s_out.at[pl.ds(o0, rows_per_tile)])
        pltpu.sync_copy(accw.at[pl.ds(r0, rows_per_tile)],
                        w_out.at[pl.ds(o0, rows_per_tile)])

    s2, w2 = body(payload, w, idx)
    return s2.reshape(2, NP_NODES, 16), w2.reshape(2, NP_NODES)


def _sc_gather(table, idx, B=1000):
    E = idx.shape[0]
    N, D = table.shape
    epw = E // NWORKERS
    assert epw % B == 0
    n_chunks = epw // B

    @functools.partial(
        pl.kernel,
        out_type=jax.ShapeDtypeStruct((E, D), F32),
        scratch_types=[
            pltpu.VMEM((B,), jnp.int32),
            pltpu.VMEM((B, D), F32),
            pltpu.SemaphoreType.DMA,
        ],
        mesh=_mesh(),
        compiler_params=pltpu.CompilerParams(use_tc_tiling_on_sc=False),
    )
    def body(table_hbm, idx_hbm, out_hbm, ibuf, rbuf, sem):
        wid = _wid()

        def chunk(j, _):
            off = wid * epw + j * B
            pltpu.sync_copy(idx_hbm.at[pl.ds(off, B)], ibuf)
            pltpu.async_copy(table_hbm.at[ibuf], rbuf, sem).wait()
            pltpu.sync_copy(rbuf, out_hbm.at[pl.ds(off, B)])
            return 0
        lax.fori_loop(0, n_chunks, chunk, 0)

    return body(table, idx)


def _sc_counts(vi, ci, E, B=2000):
    epw = E // NWORKERS
    assert epw % B == 0
    n_chunks = epw // B
    rows_per_tile = NP_NODES // 16

    @functools.partial(
        pl.kernel,
        out_type=[
            jax.ShapeDtypeStruct((2 * NP_NODES,), F32),
            jax.ShapeDtypeStruct((2 * NP_NODES,), F32),
        ],
        scratch_types=[
            pltpu.VMEM_SHARED((NP_NODES,), F32),
            pltpu.VMEM_SHARED((NP_NODES,), F32),
            pltpu.VMEM((B,), F32),
            pltpu.VMEM((B,), F32),
            pltpu.VMEM((B,), jnp.int32),
            pltpu.VMEM((B,), jnp.int32),
        ],
        mesh=_mesh(),
        compiler_params=pltpu.CompilerParams(use_tc_tiling_on_sc=False),
    )
    def body(vi_hbm, ci_hbm, cv_out, cc_out,
             accv, accc, zeros_b, ones, ivbuf, icbuf):
        cid = lax.axis_index("c")
        sid = lax.axis_index("s")
        wid = sid * 2 + cid

        def zrow(i, _):
            zeros_b[pl.ds(i * 16, 16)] = jnp.zeros((16,), F32)
            ones[pl.ds(i * 16, 16)] = jnp.ones((16,), F32)
            return 0
        lax.fori_loop(0, B // 16, zrow, 0)

        r0 = sid * rows_per_tile
        for z0 in (0, B, 2 * B, rows_per_tile - B):
            pltpu.sync_copy(zeros_b, accv.at[pl.ds(r0 + z0, B)])
            pltpu.sync_copy(zeros_b, accc.at[pl.ds(r0 + z0, B)])
        plsc.subcore_barrier()

        def chunk(j, _):
            off = wid * epw + j * B
            pltpu.sync_copy(vi_hbm.at[pl.ds(off, B)], ivbuf)
            pltpu.sync_copy(ci_hbm.at[pl.ds(off, B)], icbuf)
            pltpu.sync_copy(ones, accv.at[ivbuf], add=True)
            pltpu.sync_copy(ones, accc.at[icbuf], add=True)
            return 0
        lax.fori_loop(0, n_chunks, chunk, 0)

        plsc.subcore_barrier()
        o0 = cid * NP_NODES + r0
        pltpu.sync_copy(accv.at[pl.ds(r0, rows_per_tile)],
                        cv_out.at[pl.ds(o0, rows_per_tile)])
        pltpu.sync_copy(accc.at[pl.ds(r0, rows_per_tile)],
                        cc_out.at[pl.ds(o0, rows_per_tile)])

    cv2, cc2 = body(vi, ci)
    return cv2.reshape(2, NP_NODES), cc2.reshape(2, NP_NODES)


_DEBUG_XLA_SCATTER = False


def _linear(x, W, b):
    return x @ W + b


def _tc_edge_conv(kjf, vjf, qdf, el8, elp8, p):
    R = kjf.shape[0]
    BR = 4000
    assert R % BR == 0
    eye8 = jnp.eye(8, dtype=F32)
    WeL = jnp.kron(eye8, p['We'][:8])
    WeP = jnp.kron(eye8, p['We'][8:12])
    be8 = jnp.tile(p['be'], 8)[None, :]
    Gsum = jnp.repeat(eye8, 16, axis=0)
    Gbc = jnp.repeat(eye8, 16, axis=1)
    scale = 0.25

    def body(kj_r, vj_r, qd_r, el_r, elp_r, wel_r, wep_r, be_r, gs_r, gb_r,
             pay_r, w_r):
        e1 = jnp.dot(el_r[...], wel_r[...], preferred_element_type=F32)
        e1 += jnp.dot(elp_r[...], wep_r[...], preferred_element_type=F32)
        e1 += be_r[...]
        tq = qd_r[...] * (kj_r[...] + e1)
        alpha = jnp.dot(tq, gs_r[...], preferred_element_type=F32) * scale
        w = jnp.exp(alpha)
        wb = jnp.dot(w, gb_r[...], preferred_element_type=F32)
        pay_r[...] = wb * (vj_r[...] + e1)
        w_r[...] = w

    full = lambda shape: pl.BlockSpec(shape, lambda i: (0, 0))
    rows = lambda w_: pl.BlockSpec((BR, w_), lambda i: (i, 0))
    return pl.pallas_call(
        body,
        grid=(R // BR,),
        in_specs=[rows(128), rows(128), rows(128), rows(64), rows(32),
                  full((64, 128)), full((32, 128)), full((1, 128)),
                  full((128, 8)), full((8, 128))],
        out_specs=[rows(128), rows(8)],
        out_shape=[jax.ShapeDtypeStruct((R, 128), F32),
                   jax.ShapeDtypeStruct((R, 8), F32)],
    )(kjf, vjf, qdf, el8, elp8, WeL, WeP, be8, Gsum, Gbc)


def _tc_edge_mlp(pvf, pcf, el8, elp8, ep):
    R = pvf.shape[0]
    BR = 4000
    assert R % BR == 0
    eye8 = jnp.eye(8, dtype=F32)
    W1 = ep['edge_mlp']['W1']
    W1L = jnp.kron(eye8, W1[:8])
    W1P = jnp.kron(eye8, W1[8:12])
    i8 = jnp.eye(8, dtype=F32)
    Wpp = jnp.kron(eye8, jnp.concatenate([i8, i8], axis=0))
    b1 = jnp.tile(ep['edge_mlp']['b1'], 8)[None, :]
    W2 = jnp.kron(eye8, ep['edge_mlp']['W2'])
    b2 = jnp.tile(ep['edge_mlp']['b2'], 8)[None, :]

    def body(pv_r, pc_r, el_r, elp_r, w1l_r, w1p_r, wpp_r, b1_r, w2_r, b2_r,
             o_r):
        h = jnp.dot(el_r[...], w1l_r[...], preferred_element_type=F32)
        h += jnp.dot(elp_r[...], w1p_r[...], preferred_element_type=F32)
        pp = pv_r[...] + pc_r[...]
        h += jnp.dot(pp, wpp_r[...], preferred_element_type=F32)
        h = jnp.maximum(h + b1_r[...], 0.0)
        o = jnp.dot(h, w2_r[...], preferred_element_type=F32) + b2_r[...]
        o_r[...] = jnp.maximum(o, 0.0)

    full = lambda shape: pl.BlockSpec(shape, lambda i: (0, 0))
    rows = lambda w_: pl.BlockSpec((BR, w_), lambda i: (i, 0))
    return pl.pallas_call(
        body,
        grid=(R // BR,),
        in_specs=[rows(128), rows(128), rows(64), rows(32),
                  full((64, 64)), full((32, 64)), full((128, 64)),
                  full((1, 64)), full((64, 64)), full((1, 64))],
        out_specs=rows(64),
        out_shape=jax.ShapeDtypeStruct((R, 64), F32),
    )(pvf, pcf, el8, elp8, W1L, W1P, Wpp, b1, W2, b2)


def _conv(p, x_src, x_dst, e_src, e_dst, el8, elp8, winv_den):
    E = e_src.shape[0]
    q = _linear(x_dst, p['Wq'], p['bq'])
    k = _linear(x_src, p['Wk'], p['bk'])
    v = _linear(x_src, p['Wv'], p['bv'])
    kjf = _sc_gather(k, e_src).reshape(E // 8, 128)
    vjf = _sc_gather(v, e_src).reshape(E // 8, 128)
    qdf = _sc_gather(q, e_dst).reshape(E // 8, 128)
    pay, w8 = _tc_edge_conv(kjf, vjf, qdf, el8, elp8, p)

    Sp, Wp = _sc_scatter(pay.reshape(E, 16), w8.reshape(E), e_dst, E)
    n_dst = x_dst.shape[0]
    S = (Sp[0] + Sp[1])[:n_dst]
    Wd = (Wp[0] + Wp[1])[:n_dst]
    out = S / (jnp.maximum(Wd, 1e-16) * winv_den[:n_dst])[:, None]
    out = out + _linear(x_dst, p['Ws'], p['bs'])
    return jax.nn.relu(out)


def _mlp2(x, p):
    h = jax.nn.relu(_linear(x, p['W1'], p['b1']))
    return jax.nn.relu(_linear(h, p['W2'], p['b2']))


def kernel(var_learned_f, var_lp_f, con_learned_f, con_lp_f, edge_learned_f,
           solver_state, edge_lp_f_wo_ss, edge_index_var_con, params):
    ei = edge_index_var_con
    vi, ci = ei[0], ei[1]

    E = vi.shape[0]

    var_comb = jnp.concatenate([var_learned_f, var_lp_f], axis=1)
    con_comb = jnp.concatenate([con_learned_f, con_lp_f], axis=1)
    el8 = edge_learned_f.reshape(E // 8, 64)
    elp8 = edge_lp_f_wo_ss.reshape(E // 8, 32)

    Cvp, Ccp = _sc_counts(vi, ci, E)
    cnt_var = jnp.maximum(Cvp[0] + Cvp[1], 1.0)
    cnt_con = jnp.maximum(Ccp[0] + Ccp[1], 1.0)

    con_new = _conv(params['con_upd'], var_comb, con_comb, vi, ci, el8, elp8, cnt_con)
    con_comb2 = jnp.concatenate([con_new, con_lp_f], axis=1)
    var_new = _conv(params['var_upd'], con_comb2, var_comb, ci, vi, el8, elp8, cnt_var)
    var_comb2 = jnp.concatenate([var_new, var_lp_f], axis=1)

    ep = params['edge_upd']
    vc = _mlp2(var_comb2, ep['var_mlp'])
    cc = _mlp2(con_comb2, ep['con_mlp'])
    W1 = ep['edge_mlp']['W1']
    W1v, W1c = W1[12:20], W1[20:28]
    pv = jnp.pad(vc @ W1v, ((0, 0), (0, 8)))
    pc = jnp.pad(cc @ W1c, ((0, 0), (8, 0)))
    pvf = _sc_gather(pv, vi).reshape(E // 8, 128)
    pcf = _sc_gather(pc, ci).reshape(E // 8, 128)
    edge_new = _tc_edge_mlp(pvf, pcf, el8, elp8, ep).reshape(E, 8)
    return (var_new, con_new, edge_new)
